# baseline (speedup 1.0000x reference)
import functools

import jax
import jax.numpy as jnp
from jax import lax
from jax.experimental import pallas as pl
from jax.experimental.pallas import tpu as pltpu
from jax.experimental.pallas import tpu_sc as plsc

ROWS = 100001
DESC_DIM = 128
TSE_DIM = 64
BATCH = 4096
HIST = 50
TOTAL = BATCH * HIST

ROW_BLOCK = 2048

NUM_CORES = 2
NUM_SUBCORES = 16
NW = NUM_CORES * NUM_SUBCORES
BATCH_PER_W = BATCH // NW
CHUNK_B = 2
CHUNK_I = CHUNK_B * HIST
CHUNK_P = 104
NCHUNK = BATCH_PER_W // CHUNK_B


def _fuse_body(sem_ref, wt_ref, b_ref, idt_ref, out_ref):
    out_ref[...] = (
        jnp.dot(sem_ref[...], wt_ref[...], preferred_element_type=jnp.float32)
        + b_ref[...]
        + idt_ref[...].T
    )


def _fused_table(sem, wt, b2, idt):
    grid = (pl.cdiv(ROWS, ROW_BLOCK),)
    return pl.pallas_call(
        _fuse_body,
        grid=grid,
        in_specs=[
            pl.BlockSpec((ROW_BLOCK, DESC_DIM), lambda i: (i, 0)),
            pl.BlockSpec((DESC_DIM, TSE_DIM), lambda i: (0, 0)),
            pl.BlockSpec((1, TSE_DIM), lambda i: (0, 0)),
            pl.BlockSpec((TSE_DIM, ROW_BLOCK), lambda i: (0, i)),
        ],
        out_specs=pl.BlockSpec((ROW_BLOCK, TSE_DIM), lambda i: (i, 0)),
        out_shape=jax.ShapeDtypeStruct((ROWS, TSE_DIM), jnp.float32),
    )(sem, wt, b2, idt)


def _gather(idx3, table):
    mesh = plsc.VectorSubcoreMesh(core_axis_name="c", subcore_axis_name="s")

    @functools.partial(
        pl.kernel,
        mesh=mesh,
        compiler_params=pltpu.CompilerParams(use_tc_tiling_on_sc=False),
        out_type=jax.ShapeDtypeStruct((BATCH, HIST, TSE_DIM), jnp.float32),
        scratch_types=[
            pltpu.VMEM((NCHUNK, CHUNK_P), jnp.int32),
            pltpu.VMEM((CHUNK_P, TSE_DIM), jnp.float32),
            pltpu.VMEM((CHUNK_P, TSE_DIM), jnp.float32),
            pltpu.SemaphoreType.DMA,
            pltpu.SemaphoreType.DMA,
        ],
    )
    def k(idx_hbm, table_hbm, out_hbm, idx_v, buf0, buf1, gsem0, gsem1):
        wid = lax.axis_index("s") * NUM_CORES + lax.axis_index("c")
        bbase = wid * BATCH_PER_W
        pltpu.sync_copy(idx_hbm.at[wid], idx_v)

        pltpu.async_copy(table_hbm.at[idx_v.at[0]], buf0, gsem0)

        def write_chunk(buf, j):
            b0 = bbase + j * CHUNK_B
            pltpu.sync_copy(buf.at[pl.ds(0, HIST)], out_hbm.at[b0])
            pltpu.sync_copy(buf.at[pl.ds(HIST, HIST)], out_hbm.at[b0 + 1])

        def pair_body(p, _):
            j0 = p * 2
            pltpu.make_async_copy(table_hbm.at[idx_v.at[j0]], buf0, gsem0).wait()

            @pl.when(j0 + 1 < NCHUNK)
            def _():
                pltpu.async_copy(table_hbm.at[idx_v.at[j0 + 1]], buf1, gsem1)

            write_chunk(buf0, j0)

            @pl.when(j0 + 1 < NCHUNK)
            def _():
                pltpu.make_async_copy(
                    table_hbm.at[idx_v.at[j0 + 1]], buf1, gsem1
                ).wait()

                @pl.when(j0 + 2 < NCHUNK)
                def _():
                    pltpu.async_copy(table_hbm.at[idx_v.at[j0 + 2]], buf0, gsem0)

                write_chunk(buf1, j0 + 1)

            return 0

        lax.fori_loop(0, (NCHUNK + 1) // 2, pair_body, 0)

    return k(idx3, table)


def kernel(product_ids, semantic_table, fcn_W, fcn_b, id_table):
    wt = fcn_W.T
    b2 = fcn_b.reshape(1, TSE_DIM)
    idt = id_table.T
    table = _fused_table(semantic_table, wt, b2, idt)
    ids2 = product_ids.astype(jnp.int32).reshape(TOTAL // CHUNK_I, CHUNK_I)
    pad = jnp.zeros((TOTAL // CHUNK_I, CHUNK_P - CHUNK_I), jnp.int32)
    idx3 = jnp.concatenate([ids2, pad], axis=1).reshape(NW, NCHUNK, CHUNK_P)
    return _gather(idx3, table)


# speedup vs baseline: 1.6253x; 1.6253x over previous
import functools

import jax
import jax.numpy as jnp
from jax import lax
from jax.experimental import pallas as pl
from jax.experimental.pallas import tpu as pltpu
from jax.experimental.pallas import tpu_sc as plsc

ROWS = 100001
DESC_DIM = 128
TSE_DIM = 64
BATCH = 4096
HIST = 50
TOTAL = BATCH * HIST

ROW_BLOCK = 2048

NUM_CORES = 2
NUM_SUBCORES = 16
NW = NUM_CORES * NUM_SUBCORES
PER_W = TOTAL // NW
CHUNK = 128
NCHUNK = PER_W // CHUNK


def _fuse_body(sem_ref, wt_ref, b_ref, idt_ref, out_ref):
    out_ref[...] = (
        jnp.dot(sem_ref[...], wt_ref[...], preferred_element_type=jnp.float32)
        + b_ref[...]
        + idt_ref[...].T
    )


def _fused_table(sem, wt, b2, idt):
    grid = (pl.cdiv(ROWS, ROW_BLOCK),)
    return pl.pallas_call(
        _fuse_body,
        grid=grid,
        in_specs=[
            pl.BlockSpec((ROW_BLOCK, DESC_DIM), lambda i: (i, 0)),
            pl.BlockSpec((DESC_DIM, TSE_DIM), lambda i: (0, 0)),
            pl.BlockSpec((1, TSE_DIM), lambda i: (0, 0)),
            pl.BlockSpec((TSE_DIM, ROW_BLOCK), lambda i: (0, i)),
        ],
        out_specs=pl.BlockSpec((ROW_BLOCK, TSE_DIM), lambda i: (i, 0)),
        out_shape=jax.ShapeDtypeStruct((ROWS, TSE_DIM), jnp.float32),
    )(sem, wt, b2, idt)


def _gather(idx3, table):
    mesh = plsc.VectorSubcoreMesh(core_axis_name="c", subcore_axis_name="s")

    @functools.partial(
        pl.kernel,
        mesh=mesh,
        compiler_params=pltpu.CompilerParams(use_tc_tiling_on_sc=False),
        out_type=jax.ShapeDtypeStruct((TOTAL, TSE_DIM), jnp.float32),
        scratch_types=[
            pltpu.VMEM((NCHUNK, CHUNK), jnp.int32),
            pltpu.VMEM((CHUNK, TSE_DIM), jnp.float32),
            pltpu.VMEM((CHUNK, TSE_DIM), jnp.float32),
            pltpu.SemaphoreType.DMA,
            pltpu.SemaphoreType.DMA,
        ],
    )
    def k(idx_hbm, table_hbm, out_hbm, idx_v, buf0, buf1, gsem0, gsem1):
        wid = lax.axis_index("s") * NUM_CORES + lax.axis_index("c")
        base = wid * PER_W
        pltpu.sync_copy(idx_hbm.at[wid], idx_v)

        pltpu.async_copy(table_hbm.at[idx_v.at[0]], buf0, gsem0)

        def pair_body(p, _):
            j0 = p * 2
            pltpu.make_async_copy(table_hbm.at[idx_v.at[j0]], buf0, gsem0).wait()

            @pl.when(j0 + 1 < NCHUNK)
            def _():
                pltpu.async_copy(table_hbm.at[idx_v.at[j0 + 1]], buf1, gsem1)

            pltpu.sync_copy(buf0, out_hbm.at[pl.ds(base + j0 * CHUNK, CHUNK)])

            @pl.when(j0 + 1 < NCHUNK)
            def _():
                pltpu.make_async_copy(
                    table_hbm.at[idx_v.at[j0 + 1]], buf1, gsem1
                ).wait()

                @pl.when(j0 + 2 < NCHUNK)
                def _():
                    pltpu.async_copy(table_hbm.at[idx_v.at[j0 + 2]], buf0, gsem0)

                pltpu.sync_copy(
                    buf1, out_hbm.at[pl.ds(base + (j0 + 1) * CHUNK, CHUNK)]
                )

            return 0

        lax.fori_loop(0, (NCHUNK + 1) // 2, pair_body, 0)

    return k(idx3, table)


def kernel(product_ids, semantic_table, fcn_W, fcn_b, id_table):
    wt = fcn_W.T
    b2 = fcn_b.reshape(1, TSE_DIM)
    idt = id_table.T
    table = _fused_table(semantic_table, wt, b2, idt)
    idx3 = product_ids.astype(jnp.int32).reshape(NW, NCHUNK, CHUNK)
    out = _gather(idx3, table)
    return out.reshape(BATCH, HIST, TSE_DIM)
